# baseline (speedup 1.0000x reference)
import jax
import jax.numpy as jnp
from jax.experimental import pallas as pl

N = 10000
TM = 400


def _matmul_kernel(x_ref, w_ref, o_ref):
    o_ref[...] = jax.lax.dot_general(
        x_ref[...], w_ref[...], (((1,), (0,)), ((), ())),
        preferred_element_type=jnp.float32)


def _layer1_kernel(adj_ref, s1_ref, b1_ref, w2_ref, o_ref):
    h = jax.lax.dot_general(
        adj_ref[...], s1_ref[...], (((1,), (0,)), ((), ())),
        preferred_element_type=jnp.float32)
    h = jnp.maximum(h + b1_ref[...], 0.0)
    o_ref[...] = jax.lax.dot_general(
        h, w2_ref[...], (((1,), (0,)), ((), ())),
        preferred_element_type=jnp.float32)


def _layer2_kernel(adj_ref, s2_ref, b2_ref, wmu_ref, bmu_ref, wlv_ref,
                   blv_ref, mu_ref, lv_ref):
    h = jax.lax.dot_general(
        adj_ref[...], s2_ref[...], (((1,), (0,)), ((), ())),
        preferred_element_type=jnp.float32)
    h = jnp.maximum(h + b2_ref[...], 0.0)
    mu_ref[...] = jax.lax.dot_general(
        h, wmu_ref[...], (((1,), (0,)), ((), ())),
        preferred_element_type=jnp.float32) + bmu_ref[...]
    lv_ref[...] = jax.lax.dot_general(
        h, wlv_ref[...], (((1,), (0,)), ((), ())),
        preferred_element_type=jnp.float32) + blv_ref[...]


def kernel(x, adj, W1, b1, W2, b2, Wmu, bmu, Wlv, blv):
    n, nfeat = x.shape
    nhid = W1.shape[1]
    latent = Wmu.shape[1]
    b1r = b1.reshape(1, nhid)
    b2r = b2.reshape(1, nhid)
    bmur = bmu.reshape(1, latent)
    blvr = blv.reshape(1, latent)

    s1 = pl.pallas_call(
        _matmul_kernel,
        out_shape=jax.ShapeDtypeStruct((n, nhid), jnp.float32),
    )(x, W1)

    grid = (n // TM,)
    full = lambda i: (0, 0)
    row_tile = lambda i: (i, 0)

    s2 = pl.pallas_call(
        _layer1_kernel,
        grid=grid,
        in_specs=[
            pl.BlockSpec((TM, n), row_tile),
            pl.BlockSpec((n, nhid), full),
            pl.BlockSpec((1, nhid), full),
            pl.BlockSpec((nhid, nhid), full),
        ],
        out_specs=pl.BlockSpec((TM, nhid), row_tile),
        out_shape=jax.ShapeDtypeStruct((n, nhid), jnp.float32),
    )(adj, s1, b1r, W2)

    mu, lv = pl.pallas_call(
        _layer2_kernel,
        grid=grid,
        in_specs=[
            pl.BlockSpec((TM, n), row_tile),
            pl.BlockSpec((n, nhid), full),
            pl.BlockSpec((1, nhid), full),
            pl.BlockSpec((nhid, latent), full),
            pl.BlockSpec((1, latent), full),
            pl.BlockSpec((nhid, latent), full),
            pl.BlockSpec((1, latent), full),
        ],
        out_specs=[
            pl.BlockSpec((TM, latent), row_tile),
            pl.BlockSpec((TM, latent), row_tile),
        ],
        out_shape=[
            jax.ShapeDtypeStruct((n, latent), jnp.float32),
            jax.ShapeDtypeStruct((n, latent), jnp.float32),
        ],
    )(adj, s2, b2r, Wmu, bmur, Wlv, blvr)

    return (mu, lv)


# speedup vs baseline: 1.0039x; 1.0039x over previous
import jax
import jax.numpy as jnp
from jax.experimental import pallas as pl
from jax.experimental.pallas import tpu as pltpu

TM = 200


def _fused_kernel(x_ref, adj_ref, w1_ref, b1_ref, w2_ref, b2_ref,
                  wmu_ref, bmu_ref, wlv_ref, blv_ref,
                  mu_ref, lv_ref, s_ref):
    p = pl.program_id(0)
    i = pl.program_id(1)

    @pl.when(jnp.logical_and(p == 0, i == 0))
    def _seed():
        s_ref[0] = jax.lax.dot_general(
            x_ref[...], w1_ref[...], (((1,), (0,)), ((), ())),
            preferred_element_type=jnp.float32)

    s = s_ref[p]
    h = jax.lax.dot_general(
        adj_ref[...], s, (((1,), (0,)), ((), ())),
        preferred_element_type=jnp.float32)
    b = jnp.where(p == 0, b1_ref[...], b2_ref[...])
    h = jnp.maximum(h + b, 0.0)

    @pl.when(p == 0)
    def _phase0():
        s_ref[1, pl.ds(i * TM, TM), :] = jax.lax.dot_general(
            h, w2_ref[...], (((1,), (0,)), ((), ())),
            preferred_element_type=jnp.float32)

    @pl.when(p == 1)
    def _phase1():
        mu_ref[...] = jax.lax.dot_general(
            h, wmu_ref[...], (((1,), (0,)), ((), ())),
            preferred_element_type=jnp.float32) + bmu_ref[...]
        lv_ref[...] = jax.lax.dot_general(
            h, wlv_ref[...], (((1,), (0,)), ((), ())),
            preferred_element_type=jnp.float32) + blv_ref[...]


def kernel(x, adj, W1, b1, W2, b2, Wmu, bmu, Wlv, blv):
    n, nfeat = x.shape
    nhid = W1.shape[1]
    latent = Wmu.shape[1]

    full = lambda p, i: (0, 0)
    row_tile = lambda p, i: (i, 0)
    out_tile = lambda p, i: (jnp.where(p == 0, 0, i), 0)

    mu, lv = pl.pallas_call(
        _fused_kernel,
        grid=(2, n // TM),
        in_specs=[
            pl.BlockSpec((n, nfeat), full),
            pl.BlockSpec((TM, n), row_tile),
            pl.BlockSpec((nfeat, nhid), full),
            pl.BlockSpec((1, nhid), full),
            pl.BlockSpec((nhid, nhid), full),
            pl.BlockSpec((1, nhid), full),
            pl.BlockSpec((nhid, latent), full),
            pl.BlockSpec((1, latent), full),
            pl.BlockSpec((nhid, latent), full),
            pl.BlockSpec((1, latent), full),
        ],
        out_specs=[
            pl.BlockSpec((TM, latent), out_tile),
            pl.BlockSpec((TM, latent), out_tile),
        ],
        out_shape=[
            jax.ShapeDtypeStruct((n, latent), jnp.float32),
            jax.ShapeDtypeStruct((n, latent), jnp.float32),
        ],
        scratch_shapes=[pltpu.VMEM((2, n, nhid), jnp.float32)],
    )(x, adj, W1, b1.reshape(1, nhid), W2, b2.reshape(1, nhid),
      Wmu, bmu.reshape(1, latent), Wlv, blv.reshape(1, latent))

    return (mu, lv)


# speedup vs baseline: 1.0496x; 1.0456x over previous
import jax
import jax.numpy as jnp
from jax.experimental import pallas as pl
from jax.experimental.pallas import tpu as pltpu

TM = 400


def _fused_kernel(x_ref, adj_ref, w1_ref, b1_ref, w2_ref, b2_ref,
                  wmu_ref, bmu_ref, wlv_ref, blv_ref,
                  mu_ref, lv_ref, s_ref):
    p = pl.program_id(0)
    i = pl.program_id(1)

    @pl.when(jnp.logical_and(p == 0, i == 0))
    def _seed():
        s_ref[0] = jax.lax.dot_general(
            x_ref[...], w1_ref[...], (((1,), (0,)), ((), ())),
            preferred_element_type=jnp.float32)

    s = s_ref[p]
    h = jax.lax.dot_general(
        adj_ref[...], s, (((1,), (0,)), ((), ())),
        preferred_element_type=jnp.float32)
    b = jnp.where(p == 0, b1_ref[...], b2_ref[...])
    h = jnp.maximum(h + b, 0.0)

    @pl.when(p == 0)
    def _phase0():
        s_ref[1, pl.ds(i * TM, TM), :] = jax.lax.dot_general(
            h, w2_ref[...], (((1,), (0,)), ((), ())),
            preferred_element_type=jnp.float32)

    @pl.when(p == 1)
    def _phase1():
        mu_ref[...] = jax.lax.dot_general(
            h, wmu_ref[...], (((1,), (0,)), ((), ())),
            preferred_element_type=jnp.float32) + bmu_ref[...]
        lv_ref[...] = jax.lax.dot_general(
            h, wlv_ref[...], (((1,), (0,)), ((), ())),
            preferred_element_type=jnp.float32) + blv_ref[...]


def kernel(x, adj, W1, b1, W2, b2, Wmu, bmu, Wlv, blv):
    n, nfeat = x.shape
    nhid = W1.shape[1]
    latent = Wmu.shape[1]

    full = lambda p, i: (0, 0)
    row_tile = lambda p, i: (i, 0)
    out_tile = lambda p, i: (jnp.where(p == 0, 0, i), 0)

    mu, lv = pl.pallas_call(
        _fused_kernel,
        grid=(2, n // TM),
        in_specs=[
            pl.BlockSpec((n, nfeat), full),
            pl.BlockSpec((TM, n), row_tile),
            pl.BlockSpec((nfeat, nhid), full),
            pl.BlockSpec((1, nhid), full),
            pl.BlockSpec((nhid, nhid), full),
            pl.BlockSpec((1, nhid), full),
            pl.BlockSpec((nhid, latent), full),
            pl.BlockSpec((1, latent), full),
            pl.BlockSpec((nhid, latent), full),
            pl.BlockSpec((1, latent), full),
        ],
        out_specs=[
            pl.BlockSpec((TM, latent), out_tile),
            pl.BlockSpec((TM, latent), out_tile),
        ],
        out_shape=[
            jax.ShapeDtypeStruct((n, latent), jnp.float32),
            jax.ShapeDtypeStruct((n, latent), jnp.float32),
        ],
        scratch_shapes=[pltpu.VMEM((2, n, nhid), jnp.float32)],
    )(x, adj, W1, b1.reshape(1, nhid), W2, b2.reshape(1, nhid),
      Wmu, bmu.reshape(1, latent), Wlv, blv.reshape(1, latent))

    return (mu, lv)
